# baseline (speedup 1.0000x reference)
import functools

import jax
import jax.numpy as jnp
from jax import lax
from jax.experimental import pallas as pl
from jax.experimental.pallas import tpu as pltpu
from jax.experimental.pallas import tpu_sc as plsc

D = 128
B = 16384 * 50
NC = 2
NS = 16
NW = NC * NS
BPW = B // NW
CH = 256
NCH = BPW // CH
NBUF = 2

_mesh = plsc.VectorSubcoreMesh(core_axis_name="c", subcore_axis_name="s")


@functools.partial(
    pl.kernel,
    mesh=_mesh,
    out_type=jax.ShapeDtypeStruct((NW, NCH, CH * D), jnp.float32),
    scratch_types=[
        pltpu.VMEM((BPW,), jnp.int32),
        pltpu.VMEM((NBUF, CH * D), jnp.float32),
        pltpu.VMEM((4, D), jnp.float32),
        pltpu.SemaphoreType.DMA((NBUF,)),
    ],
    compiler_params=pltpu.CompilerParams(needs_layout_passes=False),
)
def _emb_lookup(idx_hbm, table_hbm, out_hbm, idx_v, buf_v, table_v, ssem):
    cid = lax.axis_index("c")
    sid = lax.axis_index("s")
    wid = sid * NC + cid
    pltpu.sync_copy(table_hbm, table_v)
    pltpu.sync_copy(idx_hbm.at[wid], idx_v)

    lane16 = lax.iota(jnp.int32, 16)
    cjota = [j * 16 + lane16 for j in range(D // 16)]

    def gen_chunk(c, b):
        @plsc.parallel_loop(0, CH // 16, unroll=2)
        def group(g):
            iv = idx_v[pl.ds(c * CH + g * 16, 16)]
            for i in range(16):
                tb = lax.gather(
                    iv, jnp.full((16, 1), i, jnp.int32),
                    lax.GatherDimensionNumbers(
                        offset_dims=(), collapsed_slice_dims=(0,),
                        start_index_map=(0,)),
                    slice_sizes=(1,),
                    mode=lax.GatherScatterMode.PROMISE_IN_BOUNDS)
                ob = (g * 16 + i) * D
                for j in range(D // 16):
                    v = plsc.load_gather(table_v, [tb, cjota[j]])
                    buf_v[b, pl.ds(ob + j * 16, 16)] = v

    def drain(b):
        pltpu.make_async_copy(buf_v.at[b], out_hbm.at[wid, 0], ssem.at[b]).wait()

    for b in range(NBUF):
        gen_chunk(b, b)
        pltpu.async_copy(buf_v.at[b], out_hbm.at[wid, b], ssem.at[b])

    def outer(o, carry):
        for b in range(NBUF):
            c = o * NBUF + b
            drain(b)
            gen_chunk(c, b)
            pltpu.async_copy(buf_v.at[b], out_hbm.at[wid, c], ssem.at[b])
        return carry
    lax.fori_loop(1, NCH // NBUF, outer, 0)

    for b in range(NBUF):
        drain(b)


def kernel(input, embedding_weight):
    idx = input.reshape(NW, BPW).astype(jnp.int32)
    out = _emb_lookup(idx, embedding_weight)
    return out.reshape(1, B, D)


# speedup vs baseline: 2.9761x; 2.9761x over previous
import functools

import jax
import jax.numpy as jnp
from jax import lax
from jax.experimental import pallas as pl
from jax.experimental.pallas import tpu as pltpu
from jax.experimental.pallas import tpu_sc as plsc

D = 128
B = 16384 * 50
NC = 2
NS = 16
NW = NC * NS
BPW = B // NW
QD = 4 * D
QPW = BPW // 4
CPQ = 32
NCH = QPW // CPQ
NBUF = 4

_mesh = plsc.VectorSubcoreMesh(core_axis_name="c", subcore_axis_name="s")


@functools.partial(
    pl.kernel,
    mesh=_mesh,
    out_type=jax.ShapeDtypeStruct((NW, NCH, CPQ, 4, D), jnp.float32),
    scratch_types=[
        pltpu.VMEM((4, QPW), jnp.int32),
        pltpu.VMEM((NCH, CPQ), jnp.int32),
        pltpu.VMEM((NBUF, CPQ, 4, D), jnp.float32),
        pltpu.VMEM_SHARED((256, 4, D), jnp.float32),
        pltpu.SemaphoreType.DMA((NBUF,)),
        pltpu.SemaphoreType.DMA((NBUF,)),
    ],
)
def _emb_lookup(idx_hbm, table_hbm, out_hbm, idx_v, qidx_v, buf_v, qtable,
                gsem, ssem):
    cid = lax.axis_index("c")
    sid = lax.axis_index("s")
    wid = sid * NC + cid

    for ql in range(16):
        q = sid * 16 + ql
        for k in range(4):
            r = (q >> (6 - 2 * k)) & 3
            pltpu.sync_copy(table_hbm.at[r], qtable.at[q, k])

    pltpu.sync_copy(idx_hbm.at[wid], idx_v)

    @plsc.parallel_loop(0, NCH, unroll=2)
    def _mk_qidx(c):
        for h in range(CPQ // 16):
            sl = pl.ds(c * CPQ + h * 16, 16)
            qv = ((idx_v[0, sl] * 4 + idx_v[1, sl]) * 4
                  + idx_v[2, sl]) * 4 + idx_v[3, sl]
            qidx_v[c, pl.ds(h * 16, 16)] = qv

    plsc.subcore_barrier()

    gat = [None] * NBUF
    scat = [None] * NBUF
    for c in range(NCH + 1):
        if c < NCH:
            b = c % NBUF
            if scat[b] is not None:
                scat[b].wait()
            gat[b] = pltpu.async_copy(
                qtable.at[qidx_v.at[c]], buf_v.at[b], gsem.at[b])
        if c >= 1:
            pb = (c - 1) % NBUF
            gat[pb].wait()
            scat[pb] = pltpu.async_copy(
                buf_v.at[pb], out_hbm.at[wid, c - 1], ssem.at[pb])
    for b in range(NBUF):
        if scat[b] is not None:
            scat[b].wait()


def kernel(input, embedding_weight):
    idx = input.reshape(NW, QPW, 4).astype(jnp.int32).transpose(0, 2, 1)
    out = _emb_lookup(idx, embedding_weight)
    return out.reshape(1, B, D)


# speedup vs baseline: 5.3651x; 1.8027x over previous
import functools

import jax
import jax.numpy as jnp
from jax import lax
from jax.experimental import pallas as pl
from jax.experimental.pallas import tpu as pltpu
from jax.experimental.pallas import tpu_sc as plsc

D = 128
B = 16384 * 50
NC = 2
NS = 16
NW = NC * NS
BPW = B // NW
CH = 128
NCH = BPW // CH
NBUF = 6

_mesh = plsc.VectorSubcoreMesh(core_axis_name="c", subcore_axis_name="s")


@functools.partial(
    pl.kernel,
    mesh=_mesh,
    out_type=jax.ShapeDtypeStruct((NW, NCH, CH, D), jnp.float32),
    scratch_types=[
        pltpu.VMEM((NCH, CH), jnp.int32),
        pltpu.VMEM((NBUF, CH, D), jnp.float32),
        pltpu.VMEM_SHARED((4, D), jnp.float32),
        pltpu.SemaphoreType.DMA((NBUF,)),
        pltpu.SemaphoreType.DMA((NBUF,)),
    ],
)
def _emb_lookup(idx_hbm, table_hbm, out_hbm, idx_v, buf_v, table_v, gsem, ssem):
    cid = lax.axis_index("c")
    sid = lax.axis_index("s")
    wid = sid * NC + cid
    @pl.when(sid == 0)
    def _():
        pltpu.sync_copy(table_hbm, table_v)

    pltpu.sync_copy(idx_hbm.at[wid], idx_v)
    plsc.subcore_barrier()
    gat = [None] * NBUF
    scat = [None] * NBUF
    for c in range(NCH + 1):
        if c < NCH:
            b = c % NBUF
            if scat[b] is not None:
                scat[b].wait()
            gat[b] = pltpu.async_copy(
                table_v.at[idx_v.at[c]], buf_v.at[b], gsem.at[b])
        if c >= 1:
            pb = (c - 1) % NBUF
            gat[pb].wait()
            scat[pb] = pltpu.async_copy(
                buf_v.at[pb], out_hbm.at[wid, c - 1], ssem.at[pb])
    for b in range(NBUF):
        if scat[b] is not None:
            scat[b].wait()


def kernel(input, embedding_weight):
    idx = input.reshape(NW, NCH, CH).astype(jnp.int32)
    out = _emb_lookup(idx, embedding_weight)
    return out.reshape(1, B, D)
